# baseline (speedup 1.0000x reference)
import functools

import jax
import jax.numpy as jnp
from jax import lax
from jax.experimental import pallas as pl
from jax.experimental.pallas import tpu as pltpu
from jax.experimental.pallas import tpu_sc as plsc

VOCAB = 50
BATCH = 16384
SEQ = 200

NW = 32
CPW = BATCH // NW
NBAND = SEQ // 8
NG = CPW // 16
TBL = 64


def _build():
    mesh = plsc.VectorSubcoreMesh(core_axis_name="c", subcore_axis_name="s")

    @functools.partial(
        pl.kernel,
        mesh=mesh,
        out_type=jax.ShapeDtypeStruct((BATCH,), jnp.float32),
        compiler_params=pltpu.CompilerParams(needs_layout_passes=False),
        scratch_types=[
            pltpu.VMEM((8, CPW), jnp.int32),
            pltpu.VMEM((8, CPW), jnp.int32),
            pltpu.VMEM((TBL,), jnp.float32),
            pltpu.VMEM((CPW,), jnp.float32),
            pltpu.SemaphoreType.DMA,
            pltpu.SemaphoreType.DMA,
        ],
    )
    def k(idxt_hbm, table_hbm, out_hbm, buf_a, buf_b, table_v, acc_v,
          sem_a, sem_b):
        wid = lax.axis_index("s") * 2 + lax.axis_index("c")
        col0 = wid * CPW
        pltpu.sync_copy(table_hbm, table_v)
        pltpu.async_copy(
            idxt_hbm.at[pl.ds(0, 8), pl.ds(col0, CPW)], buf_a, sem_a
        )
        for g in range(NG):
            acc_v[pl.ds(16 * g, 16)] = jnp.zeros((16,), jnp.float32)

        def compute_band(buf):
            for r in range(8):
                for g in range(NG):
                    iv = buf[r, pl.ds(16 * g, 16)]
                    val = plsc.load_gather(table_v, [iv])
                    plsc.addupdate(acc_v.at[pl.ds(16 * g, 16)], val)

        def wait_band(t, buf, sem):
            pltpu.make_async_copy(
                idxt_hbm.at[pl.ds(8 * t, 8), pl.ds(col0, CPW)], buf, sem
            ).wait()

        def start_band(t, buf, sem):
            pltpu.async_copy(
                idxt_hbm.at[pl.ds(8 * t, 8), pl.ds(col0, CPW)], buf, sem
            )

        def pair_body(p, carry):
            t = 2 * p
            wait_band(t, buf_a, sem_a)
            start_band(t + 1, buf_b, sem_b)
            compute_band(buf_a)
            wait_band(t + 1, buf_b, sem_b)
            start_band(t + 2, buf_a, sem_a)
            compute_band(buf_b)
            return carry

        lax.fori_loop(0, (NBAND - 1) // 2, pair_body, 0)
        wait_band(NBAND - 1, buf_a, sem_a)
        compute_band(buf_a)
        pltpu.sync_copy(acc_v, out_hbm.at[pl.ds(col0, CPW)])

    return k


_sc_kernel = _build()


@jax.jit
def kernel(inputs, tag_table):
    table_pad = jnp.concatenate(
        [tag_table, jnp.zeros((TBL - VOCAB,), jnp.float32)]
    )
    return _sc_kernel(inputs.T, table_pad)


# speedup vs baseline: 2.1687x; 2.1687x over previous
import functools

import jax
import jax.numpy as jnp
from jax import lax
from jax.experimental import pallas as pl
from jax.experimental.pallas import tpu as pltpu
from jax.experimental.pallas import tpu_sc as plsc

VOCAB = 50
BATCH = 16384
SEQ = 200

NW = 32
CPW = BATCH // NW
NG = CPW // 16
NGB = 4
GPB = NG // NGB
TBL = 64
ROWS_A = 96
ROWS_B = SEQ - ROWS_A


def _build():
    mesh = plsc.VectorSubcoreMesh(core_axis_name="c", subcore_axis_name="s")

    @functools.partial(
        pl.kernel,
        mesh=mesh,
        out_type=jax.ShapeDtypeStruct((BATCH,), jnp.float32),
        compiler_params=pltpu.CompilerParams(needs_layout_passes=False),
        scratch_types=[
            pltpu.VMEM((ROWS_A, CPW), jnp.int32),
            pltpu.VMEM((ROWS_B, CPW), jnp.int32),
            pltpu.VMEM((TBL,), jnp.float32),
            pltpu.VMEM((CPW,), jnp.float32),
            pltpu.SemaphoreType.DMA,
            pltpu.SemaphoreType.DMA,
        ],
    )
    def k(idxt_hbm, table_hbm, out_hbm, buf_a, buf_b, table_v, acc_v,
          sem_a, sem_b):
        wid = lax.axis_index("s") * 2 + lax.axis_index("c")
        col0 = wid * CPW
        pltpu.async_copy(
            idxt_hbm.at[pl.ds(0, ROWS_A), pl.ds(col0, CPW)], buf_a, sem_a
        )
        pltpu.async_copy(
            idxt_hbm.at[pl.ds(ROWS_A, ROWS_B), pl.ds(col0, CPW)], buf_b, sem_b
        )
        pltpu.sync_copy(table_hbm, table_v)

        def make_band_body(buf, gb):
            def band_body(t, accs):
                row0 = t * 8
                accs = list(accs)
                for r in range(8):
                    for j in range(GPB):
                        g = gb * GPB + j
                        iv = buf[row0 + r, pl.ds(16 * g, 16)]
                        val = plsc.load_gather(table_v, [iv])
                        accs[j] = accs[j] + val
                return tuple(accs)

            return band_body

        zero = jnp.zeros((16,), jnp.float32)
        pltpu.make_async_copy(
            idxt_hbm.at[pl.ds(0, ROWS_A), pl.ds(col0, CPW)], buf_a, sem_a
        ).wait()
        for gb in range(NGB):
            accs = tuple(zero for _ in range(GPB))
            accs = lax.fori_loop(0, ROWS_A // 8, make_band_body(buf_a, gb),
                                 accs)
            for j in range(GPB):
                acc_v[pl.ds(16 * (gb * GPB + j), 16)] = accs[j]
        pltpu.make_async_copy(
            idxt_hbm.at[pl.ds(ROWS_A, ROWS_B), pl.ds(col0, CPW)], buf_b, sem_b
        ).wait()
        for gb in range(NGB):
            accs = tuple(
                acc_v[pl.ds(16 * (gb * GPB + j), 16)] for j in range(GPB)
            )
            accs = lax.fori_loop(0, ROWS_B // 8, make_band_body(buf_b, gb),
                                 accs)
            for j in range(GPB):
                acc_v[pl.ds(16 * (gb * GPB + j), 16)] = accs[j]
        pltpu.sync_copy(acc_v, out_hbm.at[pl.ds(col0, CPW)])

    return k


_sc_kernel = _build()


@jax.jit
def kernel(inputs, tag_table):
    table_pad = jnp.concatenate(
        [tag_table, jnp.zeros((TBL - VOCAB,), jnp.float32)]
    )
    return _sc_kernel(inputs.T, table_pad)


# speedup vs baseline: 2.4192x; 1.1155x over previous
import functools

import jax
import jax.numpy as jnp
from jax import lax
from jax.experimental import pallas as pl
from jax.experimental.pallas import tpu as pltpu
from jax.experimental.pallas import tpu_sc as plsc

VOCAB = 50
BATCH = 16384
SEQ = 200

NW = 32
CPW = BATCH // NW
NG = CPW // 16
NGB = 4
GPB = NG // NGB
PTBL = 2560
ROWS_A = 96
ROWS_B = SEQ - ROWS_A


def _build():
    mesh = plsc.VectorSubcoreMesh(core_axis_name="c", subcore_axis_name="s")

    @functools.partial(
        pl.kernel,
        mesh=mesh,
        out_type=jax.ShapeDtypeStruct((BATCH,), jnp.float32),
        compiler_params=pltpu.CompilerParams(needs_layout_passes=False),
        scratch_types=[
            pltpu.VMEM((ROWS_A, CPW), jnp.int32),
            pltpu.VMEM((ROWS_B, CPW), jnp.int32),
            pltpu.VMEM((PTBL,), jnp.float32),
            pltpu.VMEM((CPW,), jnp.float32),
            pltpu.SemaphoreType.DMA,
            pltpu.SemaphoreType.DMA,
        ],
    )
    def k(idxt_hbm, ptable_hbm, out_hbm, buf_a, buf_b, ptable_v, acc_v,
          sem_a, sem_b):
        wid = lax.axis_index("s") * 2 + lax.axis_index("c")
        col0 = wid * CPW
        pltpu.async_copy(
            idxt_hbm.at[pl.ds(0, ROWS_A), pl.ds(col0, CPW)], buf_a, sem_a
        )
        pltpu.async_copy(
            idxt_hbm.at[pl.ds(ROWS_A, ROWS_B), pl.ds(col0, CPW)], buf_b, sem_b
        )
        pltpu.sync_copy(ptable_hbm, ptable_v)

        def make_band_body(buf, gb):
            def band_body(t, accs):
                row0 = t * 8
                accs = list(accs)
                for r in range(0, 8, 2):
                    for j in range(GPB):
                        g = gb * GPB + j
                        iv1 = buf[row0 + r, pl.ds(16 * g, 16)]
                        iv2 = buf[row0 + r + 1, pl.ds(16 * g, 16)]
                        cidx = iv1 * VOCAB + iv2
                        val = plsc.load_gather(ptable_v, [cidx])
                        accs[j] = accs[j] + val
                return tuple(accs)

            return band_body

        zero = jnp.zeros((16,), jnp.float32)
        pltpu.make_async_copy(
            idxt_hbm.at[pl.ds(0, ROWS_A), pl.ds(col0, CPW)], buf_a, sem_a
        ).wait()
        for gb in range(NGB):
            accs = tuple(zero for _ in range(GPB))
            accs = lax.fori_loop(0, ROWS_A // 8, make_band_body(buf_a, gb),
                                 accs)
            for j in range(GPB):
                acc_v[pl.ds(16 * (gb * GPB + j), 16)] = accs[j]
        pltpu.make_async_copy(
            idxt_hbm.at[pl.ds(ROWS_A, ROWS_B), pl.ds(col0, CPW)], buf_b, sem_b
        ).wait()
        for gb in range(NGB):
            accs = tuple(
                acc_v[pl.ds(16 * (gb * GPB + j), 16)] for j in range(GPB)
            )
            accs = lax.fori_loop(0, ROWS_B // 8, make_band_body(buf_b, gb),
                                 accs)
            for j in range(GPB):
                acc_v[pl.ds(16 * (gb * GPB + j), 16)] = accs[j]
        pltpu.sync_copy(acc_v, out_hbm.at[pl.ds(col0, CPW)])

    return k


_sc_kernel = _build()


@jax.jit
def kernel(inputs, tag_table):
    ptable = (tag_table[:, None] + tag_table[None, :]).reshape(-1)
    ptable = jnp.concatenate(
        [ptable, jnp.zeros((PTBL - VOCAB * VOCAB,), jnp.float32)]
    )
    return _sc_kernel(inputs.T, ptable)
